# baseline (speedup 1.0000x reference)
import functools

import jax
import jax.numpy as jnp
import numpy as np
from jax.experimental import pallas as pl
from jax.experimental.pallas import tpu as pltpu

B, L, H, D, S = 4, 2048, 16, 128, 64
T = B * L
TB = 512
NT = T // TB


def _router_body(xa_ref, xb_ref, w_ref, b_ref, p_ref, ones_ref, idx_ref,
                 loss_ref, sums_ref, cnts_ref):
    t = pl.program_id(0)

    @pl.when(t == 0)
    def _init():
        sums_ref[...] = jnp.zeros_like(sums_ref)
        cnts_ref[...] = jnp.zeros_like(cnts_ref)

    ones_ref[...] = jnp.ones_like(ones_ref)

    xt_a = jnp.swapaxes(xa_ref[...], 0, 1)
    xt_b = jnp.swapaxes(xb_ref[...], 0, 1)
    r_all = jnp.zeros((TB, H), dtype=jnp.float32)
    for h in range(H):
        xh = (xt_a if h < H // 2 else xt_b)[h % (H // 2)]
        logits = jnp.dot(xh, w_ref[h],
                         preferred_element_type=jnp.float32)
        logits = logits + b_ref[h][None, :]
        m = jnp.max(logits, axis=1, keepdims=True)
        e = jnp.exp(logits - m)
        denom = jnp.sum(e, axis=1, keepdims=True)
        score_sum = jnp.sum(e * (1.0 / denom), axis=0)
        is_max = jnp.where(logits == m, 1.0, 0.0)
        r_all = r_all + jnp.dot(is_max, p_ref[h],
                                preferred_element_type=jnp.float32)
        cnt = jnp.sum(is_max, axis=0)
        sums_ref[h, :] = sums_ref[h, :] + score_sum
        cnts_ref[h, :] = cnts_ref[h, :] + cnt

    rbits = jax.lax.bitcast_convert_type(r_all, jnp.int32)
    idx_ref[...] = (63 + 127) - (rbits >> 23)

    @pl.when(t == pl.num_programs(0) - 1)
    def _finish():
        prod = sums_ref[...] * cnts_ref[...]
        loss_ref[...] = (float(S) / (T * T)) * jnp.sum(prod, keepdims=True)


_P = np.zeros((H, S, H), dtype=np.float32)
for _h in range(H):
    _P[_h, :, _h] = 2.0 ** (63 - np.arange(S))


@functools.partial(jax.jit, static_argnames=())
def kernel(x, weight, bias):
    dtype = x.dtype
    x3 = x.reshape(T, H, D)
    wt = jnp.transpose(weight.astype(jnp.float32), (0, 2, 1))
    p = jnp.asarray(_P)

    ones_out, idx_out, loss_out = pl.pallas_call(
        _router_body,
        grid=(NT,),
        in_specs=[
            pl.BlockSpec((TB, H // 2, D), lambda t: (t, 0, 0)),
            pl.BlockSpec((TB, H // 2, D), lambda t: (t, 1, 0)),
            pl.BlockSpec((H, D, S), lambda t: (0, 0, 0)),
            pl.BlockSpec((H, S), lambda t: (0, 0)),
            pl.BlockSpec((H, S, H), lambda t: (0, 0, 0)),
        ],
        out_specs=[
            pl.BlockSpec((TB, H), lambda t: (t, 0)),
            pl.BlockSpec((TB, H), lambda t: (t, 0)),
            pl.BlockSpec((1, 1), lambda t: (0, 0)),
        ],
        out_shape=[
            jax.ShapeDtypeStruct((T, H), jnp.float32),
            jax.ShapeDtypeStruct((T, H), jnp.int32),
            jax.ShapeDtypeStruct((1, 1), jnp.float32),
        ],
        scratch_shapes=[
            pltpu.VMEM((H, S), jnp.float32),
            pltpu.VMEM((H, S), jnp.float32),
        ],
        compiler_params=pltpu.CompilerParams(
            dimension_semantics=("arbitrary",),
        ),
    )(x3.astype(jnp.float32), x3.astype(jnp.float32), wt,
      bias.astype(jnp.float32), p)

    sg = ones_out.reshape(B, L, H).astype(dtype)
    idx = idx_out.reshape(B, L, H)
    loss = loss_out[0, 0].astype(dtype)
    return (sg, idx, loss)


# speedup vs baseline: 1.0225x; 1.0225x over previous
import functools

import jax
import jax.numpy as jnp
import numpy as np
from jax.experimental import pallas as pl
from jax.experimental.pallas import tpu as pltpu

B, L, H, D, S = 4, 2048, 16, 128, 64
T = B * L
TB = 2048
NT = T // TB


def _router_body(xa_ref, xb_ref, w_ref, b_ref, p_ref, ones_ref, idx_ref,
                 loss_ref, sums_ref, cnts_ref):
    t = pl.program_id(0)

    @pl.when(t == 0)
    def _init():
        sums_ref[...] = jnp.zeros_like(sums_ref)
        cnts_ref[...] = jnp.zeros_like(cnts_ref)

    ones_ref[...] = jnp.ones_like(ones_ref)

    xt_a = jnp.swapaxes(xa_ref[...], 0, 1)
    xt_b = jnp.swapaxes(xb_ref[...], 0, 1)
    r_all = jnp.zeros((TB, H), dtype=jnp.float32)
    for h in range(H):
        xh = (xt_a if h < H // 2 else xt_b)[h % (H // 2)]
        logits = jnp.dot(xh, w_ref[h],
                         preferred_element_type=jnp.float32)
        logits = logits + b_ref[h][None, :]
        m = jnp.max(logits, axis=1, keepdims=True)
        e = jnp.exp(logits - m)
        denom = jnp.sum(e, axis=1, keepdims=True)
        score_sum = jnp.sum(e * (1.0 / denom), axis=0)
        is_max = jnp.where(logits == m, 1.0, 0.0)
        r_all = r_all + jnp.dot(is_max, p_ref[h],
                                preferred_element_type=jnp.float32)
        cnt = jnp.sum(is_max, axis=0)
        sums_ref[h, :] = sums_ref[h, :] + score_sum
        cnts_ref[h, :] = cnts_ref[h, :] + cnt

    rbits = jax.lax.bitcast_convert_type(r_all, jnp.int32)
    idx_ref[...] = (63 + 127) - (rbits >> 23)

    @pl.when(t == pl.num_programs(0) - 1)
    def _finish():
        prod = sums_ref[...] * cnts_ref[...]
        loss_ref[...] = (float(S) / (T * T)) * jnp.sum(prod, keepdims=True)


_P = np.zeros((H, S, H), dtype=np.float32)
for _h in range(H):
    _P[_h, :, _h] = 2.0 ** (63 - np.arange(S))


@functools.partial(jax.jit, static_argnames=())
def kernel(x, weight, bias):
    dtype = x.dtype
    x3 = x.reshape(T, H, D)
    wt = jnp.transpose(weight.astype(jnp.float32), (0, 2, 1))
    p = jnp.asarray(_P)

    ones_out, idx_out, loss_out = pl.pallas_call(
        _router_body,
        grid=(NT,),
        in_specs=[
            pl.BlockSpec((TB, H // 2, D), lambda t: (t, 0, 0)),
            pl.BlockSpec((TB, H // 2, D), lambda t: (t, 1, 0)),
            pl.BlockSpec((H, D, S), lambda t: (0, 0, 0)),
            pl.BlockSpec((H, S), lambda t: (0, 0)),
            pl.BlockSpec((H, S, H), lambda t: (0, 0, 0)),
        ],
        out_specs=[
            pl.BlockSpec((TB, H), lambda t: (t, 0)),
            pl.BlockSpec((TB, H), lambda t: (t, 0)),
            pl.BlockSpec((1, 1), lambda t: (0, 0)),
        ],
        out_shape=[
            jax.ShapeDtypeStruct((T, H), jnp.float32),
            jax.ShapeDtypeStruct((T, H), jnp.int32),
            jax.ShapeDtypeStruct((1, 1), jnp.float32),
        ],
        scratch_shapes=[
            pltpu.VMEM((H, S), jnp.float32),
            pltpu.VMEM((H, S), jnp.float32),
        ],
        compiler_params=pltpu.CompilerParams(
            dimension_semantics=("arbitrary",),
        ),
    )(x3.astype(jnp.float32), x3.astype(jnp.float32), wt,
      bias.astype(jnp.float32), p)

    sg = ones_out.reshape(B, L, H).astype(dtype)
    idx = idx_out.reshape(B, L, H)
    loss = loss_out[0, 0].astype(dtype)
    return (sg, idx, loss)
